# baseline (speedup 1.0000x reference)
import functools

import jax
import jax.numpy as jnp
from jax import lax
from jax.experimental import pallas as pl
from jax.experimental.pallas import tpu as pltpu
from jax.experimental.pallas import tpu_sc as plsc

B = 4
S = 8192
D = 1024
NC = 2
NS = 16
NW = NC * NS
S_PER_W = S // NW
CH = 8
CHUNK = CH * D
N_CHUNKS = S_PER_W // CH
LANES = 16


def _sc_add_kernel(x_hbm, pe_hbm, out_hbm, pe_buf, x_buf):
    cid = lax.axis_index("c")
    sid = lax.axis_index("s")
    wid = sid * NC + cid
    s_base = wid * S_PER_W

    def chunk_body(ci, carry):
        row0 = s_base + ci * CH
        pltpu.sync_copy(pe_hbm.at[pl.ds(row0 * D, CHUNK)], pe_buf)
        for b in range(B):
            off = (b * S + row0) * D
            pltpu.sync_copy(x_hbm.at[pl.ds(off, CHUNK)], x_buf)

            def vec_body(i, c2):
                v0 = i * (4 * LANES)
                for u in range(4):
                    o = v0 + u * LANES
                    x_buf[pl.ds(o, LANES)] = (
                        x_buf[pl.ds(o, LANES)] + pe_buf[pl.ds(o, LANES)]
                    )
                return c2

            lax.fori_loop(0, CHUNK // (4 * LANES), vec_body, 0)
            pltpu.sync_copy(x_buf, out_hbm.at[pl.ds(off, CHUNK)])
        return carry

    lax.fori_loop(0, N_CHUNKS, chunk_body, 0)


@jax.jit
def kernel(x, pe):
    x1 = x.reshape(-1)
    pe1 = pe.reshape(-1)
    mesh = plsc.VectorSubcoreMesh(core_axis_name="c", subcore_axis_name="s")
    out = pl.kernel(
        _sc_add_kernel,
        out_type=jax.ShapeDtypeStruct((B * S * D,), jnp.float32),
        mesh=mesh,
        scratch_types=[
            pltpu.VMEM((CHUNK,), jnp.float32),
            pltpu.VMEM((CHUNK,), jnp.float32),
        ],
    )(x1, pe1)
    return out.reshape(B, S, D)


# speedup vs baseline: 1.4927x; 1.4927x over previous
import functools

import jax
import jax.numpy as jnp
from jax import lax
from jax.experimental import pallas as pl
from jax.experimental.pallas import tpu as pltpu
from jax.experimental.pallas import tpu_sc as plsc

B = 4
S = 8192
D = 1024
NC = 2
NS = 16
NW = NC * NS
S_PER_W = S // NW
CH = 8
CHUNK = CH * D
N_CHUNKS = S_PER_W // CH
LANES = 16
UNROLL = 8


def _sc_add_kernel(x_hbm, pe_hbm, out_hbm,
                   pe0, pe1, x00, x01, x02, x03, x10, x11, x12, x13,
                   si0, si1, so0, so1):
    cid = lax.axis_index("c")
    sid = lax.axis_index("s")
    wid = sid * NC + cid
    s_base = wid * S_PER_W

    pe_buf = (pe0, pe1)
    xb = ((x00, x01, x02, x03), (x10, x11, x12, x13))
    si = (si0, si1)
    so = (so0, so1)

    def in_copies(ci, slot):
        row0 = s_base + ci * CH
        cps = [pltpu.make_async_copy(
            pe_hbm.at[pl.ds(row0 * D, CHUNK)], pe_buf[slot], si[slot])]
        for b in range(B):
            off = (b * S + row0) * D
            cps.append(pltpu.make_async_copy(
                x_hbm.at[pl.ds(off, CHUNK)], xb[slot][b], si[slot]))
        return cps

    def out_copies(ci, slot):
        row0 = s_base + ci * CH
        return [pltpu.make_async_copy(
            xb[slot][b], out_hbm.at[pl.ds((b * S + row0) * D, CHUNK)],
            so[slot]) for b in range(B)]

    def compute(slot):
        bufs = xb[slot]
        pe_r = pe_buf[slot]

        def grp(i, c2):
            base = i * (UNROLL * LANES)
            for u in range(UNROLL):
                o = base + u * LANES
                v = pe_r[pl.ds(o, LANES)]
                for b in range(B):
                    plsc.addupdate(bufs[b].at[pl.ds(o, LANES)], v)
            return c2

        lax.fori_loop(0, CHUNK // (UNROLL * LANES), grp, 0)

    for slot in range(2):
        for c in in_copies(slot, slot):
            c.start()

    def body(ci2, carry):
        for slot in range(2):
            ci = ci2 * 2 + slot
            for c in in_copies(ci, slot):
                c.wait()
            compute(slot)
            for c in out_copies(ci, slot):
                c.start()

            @pl.when(ci2 < N_CHUNKS // 2 - 1)
            def _refill():
                for c in out_copies(ci, slot):
                    c.wait()
                for c in in_copies(ci + 2, slot):
                    c.start()

        return carry

    lax.fori_loop(0, N_CHUNKS // 2, body, 0)

    for slot in range(2):
        for c in out_copies(N_CHUNKS - 2 + slot, slot):
            c.wait()


@jax.jit
def kernel(x, pe):
    x1 = x.reshape(-1)
    pe1 = pe.reshape(-1)
    mesh = plsc.VectorSubcoreMesh(core_axis_name="c", subcore_axis_name="s")
    out = pl.kernel(
        _sc_add_kernel,
        out_type=jax.ShapeDtypeStruct((B * S * D,), jnp.float32),
        mesh=mesh,
        scratch_types=(
            [pltpu.VMEM((CHUNK,), jnp.float32)] * 10
            + [pltpu.SemaphoreType.DMA] * 4
        ),
    )(x1, pe1)
    return out.reshape(B, S, D)


# speedup vs baseline: 3.8955x; 2.6097x over previous
import functools

import jax
import jax.numpy as jnp
from jax import lax
from jax.experimental import pallas as pl
from jax.experimental.pallas import tpu as pltpu
from jax.experimental.pallas import tpu_sc as plsc

B = 4
S = 8192
D = 1024
NC = 2
NS = 16
NW = NC * NS
S_PER_W = S // NW
CH = 8
N_CHUNKS = S_PER_W // CH
LANES = 16


def _sc_add_kernel(x_hbm, pe_hbm, out_hbm,
                   pe0, pe1, x00, x01, x02, x03, x10, x11, x12, x13,
                   si0, si1, so0, so1):
    cid = lax.axis_index("c")
    sid = lax.axis_index("s")
    wid = sid * NC + cid
    s_base = wid * S_PER_W

    pe_buf = (pe0, pe1)
    xb = ((x00, x01, x02, x03), (x10, x11, x12, x13))
    si = (si0, si1)
    so = (so0, so1)

    def in_copies(ci, slot):
        row0 = s_base + ci * CH
        cps = [pltpu.make_async_copy(
            pe_hbm.at[pl.ds(row0, CH)], pe_buf[slot], si[slot])]
        for b in range(B):
            cps.append(pltpu.make_async_copy(
                x_hbm.at[b, pl.ds(row0, CH)], xb[slot][b], si[slot]))
        return cps

    def out_copies(ci, slot):
        row0 = s_base + ci * CH
        return [pltpu.make_async_copy(
            xb[slot][b], out_hbm.at[b, pl.ds(row0, CH)],
            so[slot]) for b in range(B)]

    def compute(slot):
        bufs = xb[slot]
        pe_r = pe_buf[slot]

        def grp(i, c2):
            o = i * LANES
            for r in range(CH):
                v = pe_r[r, pl.ds(o, LANES)]
                for b in range(B):
                    plsc.addupdate(bufs[b].at[r, pl.ds(o, LANES)], v)
            return c2

        lax.fori_loop(0, D // LANES, grp, 0)

    for slot in range(2):
        for c in in_copies(slot, slot):
            c.start()

    def body(ci2, carry):
        for slot in range(2):
            ci = ci2 * 2 + slot
            for c in in_copies(ci, slot):
                c.wait()
            compute(slot)
            for c in out_copies(ci, slot):
                c.start()

            @pl.when(ci2 < N_CHUNKS // 2 - 1)
            def _refill():
                for c in out_copies(ci, slot):
                    c.wait()
                for c in in_copies(ci + 2, slot):
                    c.start()

        return carry

    lax.fori_loop(0, N_CHUNKS // 2, body, 0)

    for slot in range(2):
        for c in out_copies(N_CHUNKS - 2 + slot, slot):
            c.wait()


@jax.jit
def kernel(x, pe):
    mesh = plsc.VectorSubcoreMesh(core_axis_name="c", subcore_axis_name="s")
    return pl.kernel(
        _sc_add_kernel,
        out_type=jax.ShapeDtypeStruct((B, S, D), jnp.float32),
        mesh=mesh,
        compiler_params=pltpu.CompilerParams(use_tc_tiling_on_sc=True),
        scratch_types=(
            [pltpu.VMEM((CH, D), jnp.float32)] * 10
            + [pltpu.SemaphoreType.DMA] * 4
        ),
    )(x, pe)


# speedup vs baseline: 4.4337x; 1.1381x over previous
import functools

import jax
import jax.numpy as jnp
from jax import lax
from jax.experimental import pallas as pl
from jax.experimental.pallas import tpu as pltpu
from jax.experimental.pallas import tpu_sc as plsc

B = 4
S = 8192
D = 1024
NC = 2
NS = 16
NW = NC * NS
S_PER_W = S // NW
CH = 8
N_CHUNKS = S_PER_W // CH
NSLOT = 3
LANES = 16


def _sc_add_kernel(x_hbm, pe_hbm, out_hbm,
                   pe0, pe1, pe2,
                   x00, x01, x02, x03,
                   x10, x11, x12, x13,
                   x20, x21, x22, x23,
                   si0, si1, si2, so0, so1, so2):
    cid = lax.axis_index("c")
    sid = lax.axis_index("s")
    wid = sid * NC + cid
    s_base = wid * S_PER_W

    pe_buf = (pe0, pe1, pe2)
    xb = ((x00, x01, x02, x03), (x10, x11, x12, x13), (x20, x21, x22, x23))
    si = (si0, si1, si2)
    so = (so0, so1, so2)

    def in_copies(ci, slot):
        row0 = s_base + ci * CH
        cps = [pltpu.make_async_copy(
            pe_hbm.at[pl.ds(row0, CH)], pe_buf[slot], si[slot])]
        for b in range(B):
            cps.append(pltpu.make_async_copy(
                x_hbm.at[b, pl.ds(row0, CH)], xb[slot][b], si[slot]))
        return cps

    def out_copies(ci, slot):
        row0 = s_base + ci * CH
        return [pltpu.make_async_copy(
            xb[slot][b], out_hbm.at[b, pl.ds(row0, CH)],
            so[slot]) for b in range(B)]

    def start_in(ci, slot):
        for c in in_copies(ci, slot):
            c.start()

    def wait_in(ci, slot):
        for c in in_copies(ci, slot):
            c.wait()

    def start_out(ci, slot):
        for c in out_copies(ci, slot):
            c.start()

    def wait_out(ci, slot):
        for c in out_copies(ci, slot):
            c.wait()

    def compute(slot):
        bufs = xb[slot]
        pe_r = pe_buf[slot]

        def grp(i, c2):
            o = i * LANES
            for r in range(CH):
                v = pe_r[r, pl.ds(o, LANES)]
                for b in range(B):
                    plsc.addupdate(bufs[b].at[r, pl.ds(o, LANES)], v)
            return c2

        lax.fori_loop(0, D // LANES, grp, 0)

    start_in(0, 0)
    start_in(1, 1)
    wait_in(0, 0)
    compute(0)
    start_out(0, 0)
    start_in(2, 2)

    def body(i2, carry):
        for p in (1, 2, 3):
            ci = 3 * i2 + p
            cur = p % 3
            prev = (p - 1) % 3
            wait_in(ci, cur)
            compute(cur)
            start_out(ci, cur)
            wait_out(ci - 1, prev)
            start_in(ci + 2, prev)
        return carry

    lax.fori_loop(0, 9, body, 0)

    for ci in (28, 29):
        cur = ci % 3
        prev = (ci - 1) % 3
        wait_in(ci, cur)
        compute(cur)
        start_out(ci, cur)
        wait_out(ci - 1, prev)
        start_in(ci + 2, prev)
    for ci in (30, 31):
        cur = ci % 3
        wait_in(ci, cur)
        compute(cur)
        start_out(ci, cur)
    for ci in (29, 30, 31):
        wait_out(ci, ci % 3)


@jax.jit
def kernel(x, pe):
    mesh = plsc.VectorSubcoreMesh(core_axis_name="c", subcore_axis_name="s")
    return pl.kernel(
        _sc_add_kernel,
        out_type=jax.ShapeDtypeStruct((B, S, D), jnp.float32),
        mesh=mesh,
        compiler_params=pltpu.CompilerParams(use_tc_tiling_on_sc=True),
        scratch_types=(
            [pltpu.VMEM((CH, D), jnp.float32)] * (NSLOT * 5)
            + [pltpu.SemaphoreType.DMA] * (NSLOT * 2)
        ),
    )(x, pe)
